# baseline (speedup 1.0000x reference)
import functools
import jax
import jax.numpy as jnp
from jax import lax
from jax.experimental import pallas as pl
from jax.experimental.pallas import tpu as pltpu
from jax.experimental.pallas import tpu_sc as plsc

N = 50000
E = 800000
SDIM = 64
NUM_RADIAL = 32
DEPTH = 3
CUTOFF = 5.0
G = 8

NSC = 2
NSUB = 16
FH = SDIM // NSC
UNIT = 128
EPT = 51200
E_PAD = NSUB * EPT
UNITS = EPT // UNIT
KCH = 16
NCH = UNITS // KCH
NBUF = 2
ACC_ROWS = 50176
ZCH = ACC_ROWS // NSUB // 16
WPT = ACC_ROWS // NSUB
BN = 2000
BE = 4096


def _silu(v):
    return v / (1.0 + jnp.exp(-v))


def _nblock(feat):
    return pl.BlockSpec((BN, feat), lambda i: (i, 0))


def _wblock(r, cdim=SDIM):
    return pl.BlockSpec((r, cdim), lambda i: (0, 0))


def _split_spec():
    return pl.BlockSpec((NSC, BN, FH), lambda i: (0, i, 0))


def _embed_body(x_ref, emb_ref, lin_ref, s_ref, slin_ref):
    xb = x_ref[...]
    iota = lax.broadcasted_iota(jnp.int32, (BN, 128), 1)
    oh = (iota == xb).astype(jnp.float32)
    s = jnp.dot(oh, emb_ref[...], preferred_element_type=jnp.float32)
    s_ref[...] = s
    sl = jnp.dot(s, lin_ref[...], preferred_element_type=jnp.float32)
    slin_ref[0] = sl[:, :FH]
    slin_ref[1] = sl[:, FH:]


def _embed_call(x2, emb_pad, lin0):
    return pl.pallas_call(
        _embed_body,
        grid=(N // BN,),
        in_specs=[_nblock(1), _wblock(128), _wblock(SDIM)],
        out_specs=[_nblock(SDIM), _split_spec()],
        out_shape=[
            jax.ShapeDtypeStruct((N, SDIM), jnp.float32),
            jax.ShapeDtypeStruct((NSC, N, FH), jnp.float32),
        ],
    )(x2, emb_pad, lin0)


def _edge_w_body(d_ref, fW1_ref, fb1_ref, fW2_ref, fb2_ref, w_ref):
    d = d_ref[...]
    n = (lax.broadcasted_iota(jnp.int32, (BE, NUM_RADIAL), 1) + 1
         ).astype(jnp.float32)
    arg = n * (jnp.pi / CUTOFF) * d
    rbf = jnp.sqrt(2.0 / CUTOFF) * jnp.sin(arg) / d
    env = 0.5 * (jnp.cos(jnp.pi * d / CUTOFF) + 1.0)
    env = env * (d < CUTOFF).astype(jnp.float32)
    h = _silu(jnp.dot(rbf, fW1_ref[...], preferred_element_type=jnp.float32)
              + fb1_ref[...])
    w = jnp.dot(h, fW2_ref[...], preferred_element_type=jnp.float32) + fb2_ref[...]
    w = w * env
    w_ref[0] = w[:, :FH]
    w_ref[1] = w[:, FH:]


def _edge_w_call(d_p, fW1, fb1, fW2, fb2):
    grid = E_PAD // BE
    return pl.pallas_call(
        _edge_w_body,
        grid=(grid,),
        in_specs=[
            pl.BlockSpec((BE, 1), lambda i: (i, 0)),
            pl.BlockSpec((NUM_RADIAL, SDIM), lambda i: (0, 0)),
            pl.BlockSpec((1, SDIM), lambda i: (0, 0)),
            pl.BlockSpec((SDIM, SDIM), lambda i: (0, 0)),
            pl.BlockSpec((1, SDIM), lambda i: (0, 0)),
        ],
        out_specs=pl.BlockSpec((NSC, BE, FH), lambda i: (0, i, 0)),
        out_shape=jax.ShapeDtypeStruct((NSC, E_PAD, FH), jnp.float32),
    )(d_p, fW1, fb1, fW2, fb2)


def _update_body(s_ref, agg_ref, deg_ref, uW1_ref, ub1_ref, uW2_ref, ub2_ref,
                 lin_ref, s_out_ref, slin_ref):
    deg = jnp.maximum(deg_ref[...], 1.0)
    a = jnp.concatenate([agg_ref[0], agg_ref[1]], axis=-1) / deg
    h = _silu(jnp.dot(a, uW1_ref[...], preferred_element_type=jnp.float32)
              + ub1_ref[...])
    s_new = s_ref[...] + jnp.dot(h, uW2_ref[...],
                                 preferred_element_type=jnp.float32) + ub2_ref[...]
    s_out_ref[...] = s_new
    sl = jnp.dot(s_new, lin_ref[...], preferred_element_type=jnp.float32)
    slin_ref[0] = sl[:, :FH]
    slin_ref[1] = sl[:, FH:]


def _update_call(s2, agg_out, deg2, uW1, ub1, uW2, ub2, lin_next):
    return pl.pallas_call(
        _update_body,
        grid=(N // BN,),
        in_specs=[
            _nblock(SDIM),
            pl.BlockSpec((NSC, BN, FH), lambda i: (0, i, 0)),
            _nblock(1),
            _wblock(SDIM), _wblock(1), _wblock(SDIM), _wblock(1),
            _wblock(SDIM),
        ],
        out_specs=[_nblock(SDIM), _split_spec()],
        out_shape=[
            jax.ShapeDtypeStruct((N, SDIM), jnp.float32),
            jax.ShapeDtypeStruct((NSC, N, FH), jnp.float32),
        ],
    )(s2, agg_out, deg2, uW1, ub1, uW2, ub2, lin_next)


def _final_body(s_ref, batch_ref, lng_ref, lnb_ref, post_ref,
                gsum_ref, gcnt_ref):
    @pl.when(pl.program_id(0) == 0)
    def _():
        gsum_ref[...] = jnp.zeros_like(gsum_ref)
        gcnt_ref[...] = jnp.zeros_like(gcnt_ref)

    s = s_ref[...]
    mu = jnp.mean(s, axis=-1, keepdims=True)
    xc = s - mu
    var = jnp.mean(xc * xc, axis=-1, keepdims=True)
    sn = xc / jnp.sqrt(var + 1e-5) * lng_ref[...] + lnb_ref[...]
    p = jnp.dot(sn, post_ref[...], preferred_element_type=jnp.float32)
    bb = batch_ref[...]
    gio = lax.broadcasted_iota(jnp.int32, (BN, G), 1)
    oh = (gio == bb).astype(jnp.float32)
    part = lax.dot_general(oh, p, (((0,), (0,)), ((), ())),
                           preferred_element_type=jnp.float32)
    cnt = lax.dot_general(oh, jnp.ones((BN, SDIM), jnp.float32),
                          (((0,), (0,)), ((), ())),
                          preferred_element_type=jnp.float32)
    gsum_ref[...] += part
    gcnt_ref[...] += cnt


def _final_call(s2, batch2, lng, lnb, post_lin):
    return pl.pallas_call(
        _final_body,
        grid=(N // BN,),
        in_specs=[
            _nblock(SDIM), _nblock(1),
            _wblock(1), _wblock(1), _wblock(SDIM),
        ],
        out_specs=[
            pl.BlockSpec((G, SDIM), lambda i: (0, 0)),
            pl.BlockSpec((G, SDIM), lambda i: (0, 0)),
        ],
        out_shape=[
            jax.ShapeDtypeStruct((G, SDIM), jnp.float32),
            jax.ShapeDtypeStruct((G, SDIM), jnp.float32),
        ],
    )(s2, batch2, lng, lnb, post_lin)


def _head_body(gsum_ref, gcnt_ref, d1W_ref, d1b_ref, d2W_ref, d2b_ref,
               a1W_ref, a1b_ref, a2W_ref, a2b_ref, out_ref):
    y = gsum_ref[...] / jnp.maximum(gcnt_ref[...], 1.0)
    y = _silu(jnp.dot(y, d1W_ref[...], preferred_element_type=jnp.float32)
              + d1b_ref[...])
    y = jnp.dot(y, d2W_ref[...], preferred_element_type=jnp.float32) + d2b_ref[...]
    a = _silu(jnp.dot(y, a1W_ref[...], preferred_element_type=jnp.float32)
              + a1b_ref[...])
    out_ref[...] = jnp.dot(a, a2W_ref[...],
                           preferred_element_type=jnp.float32) + a2b_ref[...]


def _head_call(gsum, gcnt, d1W, d1b, d2W, d2b, a1W, a1b, a2W_pad, a2b_pad):
    return pl.pallas_call(
        _head_body,
        out_shape=jax.ShapeDtypeStruct((G, 128), jnp.float32),
    )(gsum, gcnt, d1W, d1b, d2W, d2b, a1W, a1b, a2W_pad, a2b_pad)


@functools.cache
def _mesh():
    return plsc.VectorSubcoreMesh(core_axis_name="c", subcore_axis_name="s",
                                  num_cores=NSC, num_subcores=NSUB)


def _zero_acc(acc, zb, sid, width):
    zeros16 = jnp.zeros((16,), jnp.float32)

    @pl.loop(0, 16)
    def _(e):
        for j in range(width // 16):
            zb[e, pl.ds(j * 16, 16)] = zeros16

    @pl.loop(0, ZCH)
    def _(k):
        pltpu.sync_copy(zb, acc.at[pl.ds((sid * ZCH + k) * 16, 16)])


def _deg_kernel(row2_hbm, out_hbm, rbig, onesb, zb, acc, ssem):
    c = lax.axis_index("c")
    sid = lax.axis_index("s")

    ones16 = jnp.ones((16,), jnp.float32)

    @pl.loop(0, UNIT)
    def _(e):
        onesb[e, :] = ones16

    _zero_acc(acc, zb, sid, 16)
    plsc.subcore_barrier()

    @pl.loop(0, NCH)
    def _(ch):
        p = ch % 2

        @pl.when(ch >= 2)
        def _():
            @pl.loop(0, KCH)
            def _(u):
                pltpu.make_async_copy(onesb, acc.at[rbig.at[p, 0]],
                                      ssem.at[p]).wait()

        pltpu.sync_copy(
            row2_hbm.at[pl.ds(sid * UNITS + ch * KCH, KCH)], rbig.at[p])

        @pl.loop(0, KCH)
        def _(u):
            pltpu.async_copy(onesb, acc.at[rbig.at[p, u]], ssem.at[p],
                             add=True)

    for p in range(2):
        @pl.loop(0, KCH)
        def _(u):
            pltpu.make_async_copy(onesb, acc.at[rbig.at[p, 0]],
                                  ssem.at[p]).wait()

    plsc.subcore_barrier()
    pltpu.sync_copy(acc.at[pl.ds(sid * WPT, WPT)],
                    out_hbm.at[c, pl.ds(sid * WPT, WPT)])


def _deg_call(row2):
    return pl.kernel(
        _deg_kernel,
        out_type=jax.ShapeDtypeStruct((NSC, ACC_ROWS, 16), jnp.float32),
        mesh=_mesh(),
        scratch_types=[
            pltpu.VMEM((2, KCH, UNIT), jnp.int32),
            pltpu.VMEM((UNIT, 16), jnp.float32),
            pltpu.VMEM((16, 16), jnp.float32),
            pltpu.VMEM_SHARED((ACC_ROWS, 16), jnp.float32),
            pltpu.SemaphoreType.DMA((2,)),
        ],
        compiler_params=pltpu.CompilerParams(use_tc_tiling_on_sc=False),
    )(row2)


def _msg_kernel(col2_hbm, row2_hbm, w_hbm, tab_hbm, out_hbm,
                cbig, rbig, gbuf, wbuf, zb, acc, gsem, wsem, ssem):
    c = lax.axis_index("c")
    sid = lax.axis_index("s")

    _zero_acc(acc, zb, sid, FH)
    plsc.subcore_barrier()

    tab_c = tab_hbm.at[c]
    w_c = w_hbm.at[c]

    def issue(ubase, u, b):
        pltpu.async_copy(tab_c.at[cbig.at[u]], gbuf.at[b], gsem.at[b])
        pltpu.async_copy(w_c.at[pl.ds((ubase + u) * UNIT, UNIT)],
                         wbuf.at[b], wsem.at[b])

    def drain_scatter(b, u):
        pltpu.make_async_copy(gbuf.at[b], acc.at[rbig.at[u]],
                              ssem.at[b]).wait()

    AHEAD = NBUF - 1

    @pl.loop(0, NCH)
    def _(ch):
        ubase = sid * UNITS + ch * KCH
        pltpu.sync_copy(col2_hbm.at[pl.ds(ubase, KCH)], cbig)
        pltpu.sync_copy(row2_hbm.at[pl.ds(ubase, KCH)], rbig)

        for k in range(AHEAD):
            issue(ubase, k, k)

        @pl.loop(0, KCH // NBUF)
        def _(g):
            for k in range(NBUF):
                u = g * NBUF + k
                b = k
                pltpu.make_async_copy(tab_c.at[cbig.at[0]], gbuf.at[b],
                                      gsem.at[b]).wait()
                pltpu.make_async_copy(w_c.at[pl.ds(0, UNIT)], wbuf.at[b],
                                      wsem.at[b]).wait()

                @pl.loop(0, UNIT, unroll=4)
                def _(e):
                    for j in range(FH // 16):
                        sl = pl.ds(j * 16, 16)
                        gbuf[b, e, sl] = gbuf[b, e, sl] * wbuf[b, e, sl]

                pltpu.async_copy(gbuf.at[b], acc.at[rbig.at[u]],
                                 ssem.at[b], add=True)

                b2 = (k + AHEAD) % NBUF

                @pl.when(u + AHEAD < KCH)
                def _():
                    @pl.when(u >= 1)
                    def _():
                        drain_scatter(b2, u)
                    issue(ubase, u + AHEAD, b2)

        for bb in range(NBUF):
            drain_scatter(bb, 0)

    plsc.subcore_barrier()
    pltpu.sync_copy(acc.at[pl.ds(sid * WPT, WPT)],
                    out_hbm.at[c, pl.ds(sid * WPT, WPT)])


def _msg_call(col2, row2, w, slin):
    return pl.kernel(
        _msg_kernel,
        out_type=jax.ShapeDtypeStruct((NSC, ACC_ROWS, FH), jnp.float32),
        mesh=_mesh(),
        scratch_types=[
            pltpu.VMEM((KCH, UNIT), jnp.int32),
            pltpu.VMEM((KCH, UNIT), jnp.int32),
            pltpu.VMEM((NBUF, UNIT, FH), jnp.float32),
            pltpu.VMEM((NBUF, UNIT, FH), jnp.float32),
            pltpu.VMEM((16, FH), jnp.float32),
            pltpu.VMEM_SHARED((ACC_ROWS, FH), jnp.float32),
            pltpu.SemaphoreType.DMA((NBUF,)),
            pltpu.SemaphoreType.DMA((NBUF,)),
            pltpu.SemaphoreType.DMA((NBUF,)),
        ],
        compiler_params=pltpu.CompilerParams(use_tc_tiling_on_sc=False),
    )(col2, row2, w, slin)


def kernel(x, pos, batch, edge_index, edge_weights, lig_flag, chains, params):
    del pos, lig_flag, chains
    row = edge_index[0]
    col = edge_index[1]
    row2 = jnp.pad(row, (0, E_PAD - E),
                   constant_values=N).reshape(E_PAD // UNIT, UNIT)
    col2 = jnp.pad(col, (0, E_PAD - E),
                   constant_values=0).reshape(E_PAD // UNIT, UNIT)
    d_p = jnp.pad(edge_weights, (0, E_PAD - E),
                  constant_values=1.0).reshape(E_PAD, 1)
    x2 = x.reshape(N, 1)
    batch2 = batch.reshape(N, 1)

    p = params
    emb_pad = jnp.pad(p['emb'], ((0, 128 - p['emb'].shape[0]), (0, 0)))
    layers = p['layers']

    def r1(v):
        return v.reshape(1, -1)

    s2, slin = _embed_call(x2, emb_pad, layers[0]['lin'])

    deg_out = _deg_call(row2)
    deg2 = deg_out[0, :N, 0:1]

    zlin = jnp.zeros((SDIM, SDIM), jnp.float32)
    for li, lp in enumerate(layers):
        w = _edge_w_call(d_p, lp['fW1'], r1(lp['fb1']), lp['fW2'], r1(lp['fb2']))
        agg_out = _msg_call(col2, row2, w, slin)
        lin_next = layers[li + 1]['lin'] if li + 1 < DEPTH else zlin
        s2, slin = _update_call(s2, agg_out, deg2, lp['uW1'],
                                r1(lp['ub1']), lp['uW2'], r1(lp['ub2']),
                                lin_next)

    gsum, gcnt = _final_call(s2, batch2, r1(p['ln_g']), r1(p['ln_b']),
                             p['post_lin'])

    a2W_pad = jnp.pad(p['a2W'], ((0, 0), (0, 128 - p['a2W'].shape[1])))
    a2b_pad = jnp.pad(p['a2b'], (0, 128 - p['a2b'].shape[0])).reshape(1, 128)
    out = _head_call(gsum, gcnt, p['d1W'], r1(p['d1b']), p['d2W'], r1(p['d2b']),
                     p['a1W'], r1(p['a1b']), a2W_pad, a2b_pad)
    return out[:, :1]


# speedup vs baseline: 2.6319x; 2.6319x over previous
import functools
import jax
import jax.numpy as jnp
from jax import lax
from jax.experimental import pallas as pl
from jax.experimental.pallas import tpu as pltpu
from jax.experimental.pallas import tpu_sc as plsc

N = 50000
E = 800000
SDIM = 64
NUM_RADIAL = 32
DEPTH = 3
CUTOFF = 5.0
G = 8

NSC = 2
NSUB = 16
FH = SDIM // NSC
UNIT = 128
EPT = 51200
E_PAD = NSUB * EPT
UNITS = EPT // UNIT
KCH = 16
NCH = UNITS // KCH
NBUF = 2
ACC_ROWS = 50176
ZCH = ACC_ROWS // NSUB // 16
WPT = ACC_ROWS // NSUB
BN = 2000
BE = 4096


def _silu(v):
    return v / (1.0 + jnp.exp(-v))


def _nblock(feat):
    return pl.BlockSpec((BN, feat), lambda i: (i, 0))


def _wblock(r, cdim=SDIM):
    return pl.BlockSpec((r, cdim), lambda i: (0, 0))


def _split_spec():
    return pl.BlockSpec((NSC, BN, FH), lambda i: (0, i, 0))


def _embed_body(x_ref, emb_ref, lin_ref, s_ref, slin_ref):
    xb = x_ref[...]
    iota = lax.broadcasted_iota(jnp.int32, (BN, 128), 1)
    oh = (iota == xb).astype(jnp.float32)
    s = jnp.dot(oh, emb_ref[...], preferred_element_type=jnp.float32)
    s_ref[...] = s
    sl = jnp.dot(s, lin_ref[...], preferred_element_type=jnp.float32)
    slin_ref[0] = sl[:, :FH]
    slin_ref[1] = sl[:, FH:]


def _embed_call(x2, emb_pad, lin0):
    return pl.pallas_call(
        _embed_body,
        grid=(N // BN,),
        in_specs=[_nblock(1), _wblock(128), _wblock(SDIM)],
        out_specs=[_nblock(SDIM), _split_spec()],
        out_shape=[
            jax.ShapeDtypeStruct((N, SDIM), jnp.float32),
            jax.ShapeDtypeStruct((NSC, N, FH), jnp.float32),
        ],
    )(x2, emb_pad, lin0)


BQ = BE // 4


def _edge_w_body(d_ref, rmat_ref, W1_ref, b1_ref, W2a_ref, b2a_ref,
                 W2b_ref, b2b_ref, w_ref):
    dg = d_ref[...]
    dd = jnp.dot(dg, rmat_ref[...], preferred_element_type=jnp.float32)
    n = (lax.broadcasted_iota(jnp.int32, (BQ, 128), 1) % NUM_RADIAL + 1
         ).astype(jnp.float32)
    rbf = jnp.sqrt(2.0 / CUTOFF) * jnp.sin(n * (jnp.pi / CUTOFF) * dd) / dd
    envg = 0.5 * (jnp.cos(jnp.pi * dg / CUTOFF) + 1.0)
    envg = envg * (dg < CUTOFF).astype(jnp.float32)
    env = jnp.dot(envg, rmat_ref[...], preferred_element_type=jnp.float32)
    h = _silu(jnp.dot(rbf, W1_ref[...], preferred_element_type=jnp.float32)
              + b1_ref[...])
    wa = jnp.dot(h, W2a_ref[...], preferred_element_type=jnp.float32) + b2a_ref[...]
    wb = jnp.dot(h, W2b_ref[...], preferred_element_type=jnp.float32) + b2b_ref[...]
    w_ref[0] = wa * env
    w_ref[1] = wb * env


def _edge_w_call(d4, rmat, W1blk, b1t, W2a, b2a, W2b, b2b):
    grid = E_PAD // BE
    return pl.pallas_call(
        _edge_w_body,
        grid=(grid,),
        in_specs=[
            pl.BlockSpec((BQ, 4), lambda i: (i, 0)),
            pl.BlockSpec((4, 128), lambda i: (0, 0)),
            pl.BlockSpec((128, 256), lambda i: (0, 0)),
            pl.BlockSpec((1, 256), lambda i: (0, 0)),
            pl.BlockSpec((256, 128), lambda i: (0, 0)),
            pl.BlockSpec((1, 128), lambda i: (0, 0)),
            pl.BlockSpec((256, 128), lambda i: (0, 0)),
            pl.BlockSpec((1, 128), lambda i: (0, 0)),
        ],
        out_specs=pl.BlockSpec((NSC, BQ, 128), lambda i: (0, i, 0)),
        out_shape=jax.ShapeDtypeStruct((NSC, E_PAD // 4, 128), jnp.float32),
    )(d4, rmat, W1blk, b1t, W2a, b2a, W2b, b2b)


def _update_body(s_ref, agg_ref, deg_ref, uW1_ref, ub1_ref, uW2_ref, ub2_ref,
                 lin_ref, s_out_ref, slin_ref):
    deg = jnp.maximum(deg_ref[...], 1.0)
    a = jnp.concatenate([agg_ref[0], agg_ref[1]], axis=-1) / deg
    h = _silu(jnp.dot(a, uW1_ref[...], preferred_element_type=jnp.float32)
              + ub1_ref[...])
    s_new = s_ref[...] + jnp.dot(h, uW2_ref[...],
                                 preferred_element_type=jnp.float32) + ub2_ref[...]
    s_out_ref[...] = s_new
    sl = jnp.dot(s_new, lin_ref[...], preferred_element_type=jnp.float32)
    slin_ref[0] = sl[:, :FH]
    slin_ref[1] = sl[:, FH:]


def _update_call(s2, agg_out, deg2, uW1, ub1, uW2, ub2, lin_next):
    return pl.pallas_call(
        _update_body,
        grid=(N // BN,),
        in_specs=[
            _nblock(SDIM),
            pl.BlockSpec((NSC, BN, FH), lambda i: (0, i, 0)),
            _nblock(1),
            _wblock(SDIM), _wblock(1), _wblock(SDIM), _wblock(1),
            _wblock(SDIM),
        ],
        out_specs=[_nblock(SDIM), _split_spec()],
        out_shape=[
            jax.ShapeDtypeStruct((N, SDIM), jnp.float32),
            jax.ShapeDtypeStruct((NSC, N, FH), jnp.float32),
        ],
    )(s2, agg_out, deg2, uW1, ub1, uW2, ub2, lin_next)


def _final_body(s_ref, batch_ref, lng_ref, lnb_ref, post_ref,
                gsum_ref, gcnt_ref):
    @pl.when(pl.program_id(0) == 0)
    def _():
        gsum_ref[...] = jnp.zeros_like(gsum_ref)
        gcnt_ref[...] = jnp.zeros_like(gcnt_ref)

    s = s_ref[...]
    mu = jnp.mean(s, axis=-1, keepdims=True)
    xc = s - mu
    var = jnp.mean(xc * xc, axis=-1, keepdims=True)
    sn = xc / jnp.sqrt(var + 1e-5) * lng_ref[...] + lnb_ref[...]
    p = jnp.dot(sn, post_ref[...], preferred_element_type=jnp.float32)
    bb = batch_ref[...]
    gio = lax.broadcasted_iota(jnp.int32, (BN, G), 1)
    oh = (gio == bb).astype(jnp.float32)
    part = lax.dot_general(oh, p, (((0,), (0,)), ((), ())),
                           preferred_element_type=jnp.float32)
    cnt = lax.dot_general(oh, jnp.ones((BN, SDIM), jnp.float32),
                          (((0,), (0,)), ((), ())),
                          preferred_element_type=jnp.float32)
    gsum_ref[...] += part
    gcnt_ref[...] += cnt


def _final_call(s2, batch2, lng, lnb, post_lin):
    return pl.pallas_call(
        _final_body,
        grid=(N // BN,),
        in_specs=[
            _nblock(SDIM), _nblock(1),
            _wblock(1), _wblock(1), _wblock(SDIM),
        ],
        out_specs=[
            pl.BlockSpec((G, SDIM), lambda i: (0, 0)),
            pl.BlockSpec((G, SDIM), lambda i: (0, 0)),
        ],
        out_shape=[
            jax.ShapeDtypeStruct((G, SDIM), jnp.float32),
            jax.ShapeDtypeStruct((G, SDIM), jnp.float32),
        ],
    )(s2, batch2, lng, lnb, post_lin)


def _head_body(gsum_ref, gcnt_ref, d1W_ref, d1b_ref, d2W_ref, d2b_ref,
               a1W_ref, a1b_ref, a2W_ref, a2b_ref, out_ref):
    y = gsum_ref[...] / jnp.maximum(gcnt_ref[...], 1.0)
    y = _silu(jnp.dot(y, d1W_ref[...], preferred_element_type=jnp.float32)
              + d1b_ref[...])
    y = jnp.dot(y, d2W_ref[...], preferred_element_type=jnp.float32) + d2b_ref[...]
    a = _silu(jnp.dot(y, a1W_ref[...], preferred_element_type=jnp.float32)
              + a1b_ref[...])
    out_ref[...] = jnp.dot(a, a2W_ref[...],
                           preferred_element_type=jnp.float32) + a2b_ref[...]


def _head_call(gsum, gcnt, d1W, d1b, d2W, d2b, a1W, a1b, a2W_pad, a2b_pad):
    return pl.pallas_call(
        _head_body,
        out_shape=jax.ShapeDtypeStruct((G, 128), jnp.float32),
    )(gsum, gcnt, d1W, d1b, d2W, d2b, a1W, a1b, a2W_pad, a2b_pad)


@functools.cache
def _mesh():
    return plsc.VectorSubcoreMesh(core_axis_name="c", subcore_axis_name="s",
                                  num_cores=NSC, num_subcores=NSUB)


def _zero_acc(acc, zb, sid, width):
    zeros16 = jnp.zeros((16,), jnp.float32)

    @pl.loop(0, 16)
    def _(e):
        for j in range(width // 16):
            zb[e, pl.ds(j * 16, 16)] = zeros16

    @pl.loop(0, ZCH)
    def _(k):
        pltpu.sync_copy(zb, acc.at[pl.ds((sid * ZCH + k) * 16, 16)])


def _deg_kernel(row2_hbm, out_hbm, rbig, onesb, zb, acc, ssem):
    c = lax.axis_index("c")
    sid = lax.axis_index("s")

    ones16 = jnp.ones((16,), jnp.float32)

    @pl.loop(0, UNIT)
    def _(e):
        onesb[e, :] = ones16

    _zero_acc(acc, zb, sid, 16)
    plsc.subcore_barrier()

    @pl.loop(0, NCH)
    def _(ch):
        p = ch % 2

        @pl.when(ch >= 2)
        def _():
            @pl.loop(0, KCH)
            def _(u):
                pltpu.make_async_copy(onesb, acc.at[rbig.at[p, 0]],
                                      ssem.at[p]).wait()

        pltpu.sync_copy(
            row2_hbm.at[pl.ds(sid * UNITS + ch * KCH, KCH)], rbig.at[p])

        @pl.loop(0, KCH)
        def _(u):
            pltpu.async_copy(onesb, acc.at[rbig.at[p, u]], ssem.at[p],
                             add=True)

    for p in range(2):
        @pl.loop(0, KCH)
        def _(u):
            pltpu.make_async_copy(onesb, acc.at[rbig.at[p, 0]],
                                  ssem.at[p]).wait()

    plsc.subcore_barrier()
    pltpu.sync_copy(acc.at[pl.ds(sid * WPT, WPT)],
                    out_hbm.at[c, pl.ds(sid * WPT, WPT)])


def _deg_call(row2):
    return pl.kernel(
        _deg_kernel,
        out_type=jax.ShapeDtypeStruct((NSC, ACC_ROWS, 16), jnp.float32),
        mesh=_mesh(),
        scratch_types=[
            pltpu.VMEM((2, KCH, UNIT), jnp.int32),
            pltpu.VMEM((UNIT, 16), jnp.float32),
            pltpu.VMEM((16, 16), jnp.float32),
            pltpu.VMEM_SHARED((ACC_ROWS, 16), jnp.float32),
            pltpu.SemaphoreType.DMA((2,)),
        ],
        compiler_params=pltpu.CompilerParams(use_tc_tiling_on_sc=False),
    )(row2)


def _msg_kernel(col2_hbm, row2_hbm, w_hbm, tab_hbm, out_hbm,
                cbig, rbig, gbuf, wbuf, zb, acc, gsem, wsem, ssem):
    c = lax.axis_index("c")
    sid = lax.axis_index("s")

    _zero_acc(acc, zb, sid, FH)
    plsc.subcore_barrier()

    tab_c = tab_hbm.at[c]
    w_c = w_hbm.at[c]

    WROWS = UNIT * FH // 128

    def issue(ubase, u, b):
        pltpu.async_copy(tab_c.at[cbig.at[u]], gbuf.at[b], gsem.at[b])
        pltpu.async_copy(w_c.at[pl.ds((ubase + u) * WROWS, WROWS)],
                         wbuf.at[b], wsem.at[b])

    def drain_scatter(b, u):
        pltpu.make_async_copy(gbuf.at[b], acc.at[rbig.at[u]],
                              ssem.at[b]).wait()

    AHEAD = NBUF - 1

    @pl.loop(0, NCH)
    def _(ch):
        ubase = sid * UNITS + ch * KCH
        pltpu.sync_copy(col2_hbm.at[pl.ds(ubase, KCH)], cbig)
        pltpu.sync_copy(row2_hbm.at[pl.ds(ubase, KCH)], rbig)

        for k in range(AHEAD):
            issue(ubase, k, k)

        @pl.loop(0, KCH // NBUF)
        def _(g):
            for k in range(NBUF):
                u = g * NBUF + k
                b = k
                pltpu.make_async_copy(tab_c.at[cbig.at[0]], gbuf.at[b],
                                      gsem.at[b]).wait()
                pltpu.make_async_copy(w_c.at[pl.ds(0, WROWS)], wbuf.at[b],
                                      wsem.at[b]).wait()

                @pl.loop(0, WROWS, unroll=2)
                def _(r):
                    for j in range(8):
                        e = 4 * r + j // 2
                        gsl = pl.ds((j % 2) * 16, 16)
                        wsl = pl.ds(j * 16, 16)
                        gbuf[b, e, gsl] = gbuf[b, e, gsl] * wbuf[b, r, wsl]

                pltpu.async_copy(gbuf.at[b], acc.at[rbig.at[u]],
                                 ssem.at[b], add=True)

                b2 = (k + AHEAD) % NBUF

                @pl.when(u + AHEAD < KCH)
                def _():
                    @pl.when(u >= 1)
                    def _():
                        drain_scatter(b2, u)
                    issue(ubase, u + AHEAD, b2)

        for bb in range(NBUF):
            drain_scatter(bb, 0)

    plsc.subcore_barrier()
    pltpu.sync_copy(acc.at[pl.ds(sid * WPT, WPT)],
                    out_hbm.at[c, pl.ds(sid * WPT, WPT)])


def _msg_call(col2, row2, w, slin):
    return pl.kernel(
        _msg_kernel,
        out_type=jax.ShapeDtypeStruct((NSC, ACC_ROWS, FH), jnp.float32),
        mesh=_mesh(),
        scratch_types=[
            pltpu.VMEM((KCH, UNIT), jnp.int32),
            pltpu.VMEM((KCH, UNIT), jnp.int32),
            pltpu.VMEM((NBUF, UNIT, FH), jnp.float32),
            pltpu.VMEM((NBUF, UNIT * FH // 128, 128), jnp.float32),
            pltpu.VMEM((16, FH), jnp.float32),
            pltpu.VMEM_SHARED((ACC_ROWS, FH), jnp.float32),
            pltpu.SemaphoreType.DMA((NBUF,)),
            pltpu.SemaphoreType.DMA((NBUF,)),
            pltpu.SemaphoreType.DMA((NBUF,)),
        ],
        compiler_params=pltpu.CompilerParams(use_tc_tiling_on_sc=False),
    )(col2, row2, w, slin)


def kernel(x, pos, batch, edge_index, edge_weights, lig_flag, chains, params):
    del pos, lig_flag, chains
    row = edge_index[0]
    col = edge_index[1]
    row2 = jnp.pad(row, (0, E_PAD - E),
                   constant_values=N).reshape(E_PAD // UNIT, UNIT)
    col2 = jnp.pad(col, (0, E_PAD - E),
                   constant_values=0).reshape(E_PAD // UNIT, UNIT)
    d4 = jnp.pad(edge_weights, (0, E_PAD - E),
                 constant_values=1.0).reshape(E_PAD // 4, 4)
    x2 = x.reshape(N, 1)
    batch2 = batch.reshape(N, 1)

    p = params
    emb_pad = jnp.pad(p['emb'], ((0, 128 - p['emb'].shape[0]), (0, 0)))
    layers = p['layers']

    def r1(v):
        return v.reshape(1, -1)

    s2, slin = _embed_call(x2, emb_pad, layers[0]['lin'])

    deg_out = _deg_call(row2)
    deg2 = deg_out[0, :N, 0:1]

    rmat = jnp.repeat(jnp.eye(4, dtype=jnp.float32), NUM_RADIAL, axis=1)
    blkdiag = lambda m: jnp.kron(jnp.eye(4, dtype=jnp.float32), m)

    zlin = jnp.zeros((SDIM, SDIM), jnp.float32)
    for li, lp in enumerate(layers):
        w = _edge_w_call(
            d4, rmat,
            blkdiag(lp['fW1']), r1(jnp.tile(lp['fb1'], 4)),
            blkdiag(lp['fW2'][:, :FH]), r1(jnp.tile(lp['fb2'][:FH], 4)),
            blkdiag(lp['fW2'][:, FH:]), r1(jnp.tile(lp['fb2'][FH:], 4)))
        agg_out = _msg_call(col2, row2, w, slin)
        lin_next = layers[li + 1]['lin'] if li + 1 < DEPTH else zlin
        s2, slin = _update_call(s2, agg_out, deg2, lp['uW1'],
                                r1(lp['ub1']), lp['uW2'], r1(lp['ub2']),
                                lin_next)

    gsum, gcnt = _final_call(s2, batch2, r1(p['ln_g']), r1(p['ln_b']),
                             p['post_lin'])

    a2W_pad = jnp.pad(p['a2W'], ((0, 0), (0, 128 - p['a2W'].shape[1])))
    a2b_pad = jnp.pad(p['a2b'], (0, 128 - p['a2b'].shape[0])).reshape(1, 128)
    out = _head_call(gsum, gcnt, p['d1W'], r1(p['d1b']), p['d2W'], r1(p['d2b']),
                     p['a1W'], r1(p['a1b']), a2W_pad, a2b_pad)
    return out[:, :1]


# speedup vs baseline: 3.2933x; 1.2513x over previous
import functools
import jax
import jax.numpy as jnp
from jax import lax
from jax.experimental import pallas as pl
from jax.experimental.pallas import tpu as pltpu
from jax.experimental.pallas import tpu_sc as plsc

N = 50000
E = 800000
SDIM = 64
NUM_RADIAL = 32
DEPTH = 3
CUTOFF = 5.0
G = 8

NSC = 2
NSUB = 16
FH = SDIM // NSC
UNIT = 64
EPT = 51200
E_PAD = NSUB * EPT
UNITS = EPT // UNIT
KCH = 16
NCH = UNITS // KCH
NBUF = 5
ACC_ROWS = 50176
ZCH = ACC_ROWS // NSUB // 16
WPT = ACC_ROWS // NSUB
BN = 2000
BE = 4096


def _silu(v):
    return v / (1.0 + jnp.exp(-v))


def _nblock(feat):
    return pl.BlockSpec((BN, feat), lambda i: (i, 0))


def _wblock(r, cdim=SDIM):
    return pl.BlockSpec((r, cdim), lambda i: (0, 0))


def _split_spec():
    return pl.BlockSpec((NSC, BN, FH), lambda i: (0, i, 0))


def _embed_body(x_ref, emb_ref, lin_ref, s_ref, slin_ref):
    xb = x_ref[...]
    iota = lax.broadcasted_iota(jnp.int32, (BN, 128), 1)
    oh = (iota == xb).astype(jnp.float32)
    s = jnp.dot(oh, emb_ref[...], preferred_element_type=jnp.float32)
    s_ref[...] = s
    sl = jnp.dot(s, lin_ref[...], preferred_element_type=jnp.float32)
    slin_ref[0] = sl[:, :FH]
    slin_ref[1] = sl[:, FH:]


def _embed_call(x2, emb_pad, lin0):
    return pl.pallas_call(
        _embed_body,
        grid=(N // BN,),
        in_specs=[_nblock(1), _wblock(128), _wblock(SDIM)],
        out_specs=[_nblock(SDIM), _split_spec()],
        out_shape=[
            jax.ShapeDtypeStruct((N, SDIM), jnp.float32),
            jax.ShapeDtypeStruct((NSC, N, FH), jnp.float32),
        ],
    )(x2, emb_pad, lin0)


BQ = BE // 4


def _edge_w_body(d_ref, rmat_ref, W1_ref, b1_ref, W2a_ref, b2a_ref,
                 W2b_ref, b2b_ref, w_ref):
    dg = d_ref[...]
    dd = jnp.dot(dg, rmat_ref[...], preferred_element_type=jnp.float32)
    n = (lax.broadcasted_iota(jnp.int32, (BQ, 128), 1) % NUM_RADIAL + 1
         ).astype(jnp.float32)
    rbf = jnp.sqrt(2.0 / CUTOFF) * jnp.sin(n * (jnp.pi / CUTOFF) * dd) / dd
    envg = 0.5 * (jnp.cos(jnp.pi * dg / CUTOFF) + 1.0)
    envg = envg * (dg < CUTOFF).astype(jnp.float32)
    env = jnp.dot(envg, rmat_ref[...], preferred_element_type=jnp.float32)
    h = _silu(jnp.dot(rbf, W1_ref[...], preferred_element_type=jnp.float32)
              + b1_ref[...])
    wa = jnp.dot(h, W2a_ref[...], preferred_element_type=jnp.float32) + b2a_ref[...]
    wb = jnp.dot(h, W2b_ref[...], preferred_element_type=jnp.float32) + b2b_ref[...]
    w_ref[0] = wa * env
    w_ref[1] = wb * env


def _edge_w_call(d4, rmat, W1blk, b1t, W2a, b2a, W2b, b2b):
    grid = E_PAD // BE
    return pl.pallas_call(
        _edge_w_body,
        grid=(grid,),
        in_specs=[
            pl.BlockSpec((BQ, 4), lambda i: (i, 0)),
            pl.BlockSpec((4, 128), lambda i: (0, 0)),
            pl.BlockSpec((128, 256), lambda i: (0, 0)),
            pl.BlockSpec((1, 256), lambda i: (0, 0)),
            pl.BlockSpec((256, 128), lambda i: (0, 0)),
            pl.BlockSpec((1, 128), lambda i: (0, 0)),
            pl.BlockSpec((256, 128), lambda i: (0, 0)),
            pl.BlockSpec((1, 128), lambda i: (0, 0)),
        ],
        out_specs=pl.BlockSpec((NSC, BQ, 128), lambda i: (0, i, 0)),
        out_shape=jax.ShapeDtypeStruct((NSC, E_PAD // 4, 128), jnp.float32),
    )(d4, rmat, W1blk, b1t, W2a, b2a, W2b, b2b)


def _update_body(s_ref, agg_ref, deg_ref, uW1_ref, ub1_ref, uW2_ref, ub2_ref,
                 lin_ref, s_out_ref, slin_ref):
    deg = jnp.maximum(deg_ref[...], 1.0)
    a = jnp.concatenate([agg_ref[0], agg_ref[1]], axis=-1) / deg
    h = _silu(jnp.dot(a, uW1_ref[...], preferred_element_type=jnp.float32)
              + ub1_ref[...])
    s_new = s_ref[...] + jnp.dot(h, uW2_ref[...],
                                 preferred_element_type=jnp.float32) + ub2_ref[...]
    s_out_ref[...] = s_new
    sl = jnp.dot(s_new, lin_ref[...], preferred_element_type=jnp.float32)
    slin_ref[0] = sl[:, :FH]
    slin_ref[1] = sl[:, FH:]


def _update_call(s2, agg_out, deg2, uW1, ub1, uW2, ub2, lin_next):
    return pl.pallas_call(
        _update_body,
        grid=(N // BN,),
        in_specs=[
            _nblock(SDIM),
            pl.BlockSpec((NSC, BN, FH), lambda i: (0, i, 0)),
            _nblock(1),
            _wblock(SDIM), _wblock(1), _wblock(SDIM), _wblock(1),
            _wblock(SDIM),
        ],
        out_specs=[_nblock(SDIM), _split_spec()],
        out_shape=[
            jax.ShapeDtypeStruct((N, SDIM), jnp.float32),
            jax.ShapeDtypeStruct((NSC, N, FH), jnp.float32),
        ],
    )(s2, agg_out, deg2, uW1, ub1, uW2, ub2, lin_next)


def _final_body(s_ref, batch_ref, lng_ref, lnb_ref, post_ref,
                gsum_ref, gcnt_ref):
    @pl.when(pl.program_id(0) == 0)
    def _():
        gsum_ref[...] = jnp.zeros_like(gsum_ref)
        gcnt_ref[...] = jnp.zeros_like(gcnt_ref)

    s = s_ref[...]
    mu = jnp.mean(s, axis=-1, keepdims=True)
    xc = s - mu
    var = jnp.mean(xc * xc, axis=-1, keepdims=True)
    sn = xc / jnp.sqrt(var + 1e-5) * lng_ref[...] + lnb_ref[...]
    p = jnp.dot(sn, post_ref[...], preferred_element_type=jnp.float32)
    bb = batch_ref[...]
    gio = lax.broadcasted_iota(jnp.int32, (BN, G), 1)
    oh = (gio == bb).astype(jnp.float32)
    part = lax.dot_general(oh, p, (((0,), (0,)), ((), ())),
                           preferred_element_type=jnp.float32)
    cnt = lax.dot_general(oh, jnp.ones((BN, SDIM), jnp.float32),
                          (((0,), (0,)), ((), ())),
                          preferred_element_type=jnp.float32)
    gsum_ref[...] += part
    gcnt_ref[...] += cnt


def _final_call(s2, batch2, lng, lnb, post_lin):
    return pl.pallas_call(
        _final_body,
        grid=(N // BN,),
        in_specs=[
            _nblock(SDIM), _nblock(1),
            _wblock(1), _wblock(1), _wblock(SDIM),
        ],
        out_specs=[
            pl.BlockSpec((G, SDIM), lambda i: (0, 0)),
            pl.BlockSpec((G, SDIM), lambda i: (0, 0)),
        ],
        out_shape=[
            jax.ShapeDtypeStruct((G, SDIM), jnp.float32),
            jax.ShapeDtypeStruct((G, SDIM), jnp.float32),
        ],
    )(s2, batch2, lng, lnb, post_lin)


def _head_body(gsum_ref, gcnt_ref, d1W_ref, d1b_ref, d2W_ref, d2b_ref,
               a1W_ref, a1b_ref, a2W_ref, a2b_ref, out_ref):
    y = gsum_ref[...] / jnp.maximum(gcnt_ref[...], 1.0)
    y = _silu(jnp.dot(y, d1W_ref[...], preferred_element_type=jnp.float32)
              + d1b_ref[...])
    y = jnp.dot(y, d2W_ref[...], preferred_element_type=jnp.float32) + d2b_ref[...]
    a = _silu(jnp.dot(y, a1W_ref[...], preferred_element_type=jnp.float32)
              + a1b_ref[...])
    out_ref[...] = jnp.dot(a, a2W_ref[...],
                           preferred_element_type=jnp.float32) + a2b_ref[...]


def _head_call(gsum, gcnt, d1W, d1b, d2W, d2b, a1W, a1b, a2W_pad, a2b_pad):
    return pl.pallas_call(
        _head_body,
        out_shape=jax.ShapeDtypeStruct((G, 128), jnp.float32),
    )(gsum, gcnt, d1W, d1b, d2W, d2b, a1W, a1b, a2W_pad, a2b_pad)


@functools.cache
def _mesh():
    return plsc.VectorSubcoreMesh(core_axis_name="c", subcore_axis_name="s",
                                  num_cores=NSC, num_subcores=NSUB)


def _zero_acc(acc, zb, sid, width):
    zeros16 = jnp.zeros((16,), jnp.float32)

    @pl.loop(0, 16)
    def _(e):
        for j in range(width // 16):
            zb[e, pl.ds(j * 16, 16)] = zeros16

    @pl.loop(0, ZCH)
    def _(k):
        pltpu.sync_copy(zb, acc.at[pl.ds((sid * ZCH + k) * 16, 16)])


def _deg_kernel(row2_hbm, out_hbm, rbig, onesb, zb, acc, ssem):
    c = lax.axis_index("c")
    sid = lax.axis_index("s")

    ones16 = jnp.ones((16,), jnp.float32)

    @pl.loop(0, UNIT)
    def _(e):
        onesb[e, :] = ones16

    _zero_acc(acc, zb, sid, 16)
    plsc.subcore_barrier()

    @pl.loop(0, NCH)
    def _(ch):
        p = ch % 2

        @pl.when(ch >= 2)
        def _():
            @pl.loop(0, KCH)
            def _(u):
                pltpu.make_async_copy(onesb, acc.at[rbig.at[p, 0]],
                                      ssem.at[p]).wait()

        pltpu.sync_copy(
            row2_hbm.at[pl.ds(sid * UNITS + ch * KCH, KCH)], rbig.at[p])

        @pl.loop(0, KCH)
        def _(u):
            pltpu.async_copy(onesb, acc.at[rbig.at[p, u]], ssem.at[p],
                             add=True)

    for p in range(2):
        @pl.loop(0, KCH)
        def _(u):
            pltpu.make_async_copy(onesb, acc.at[rbig.at[p, 0]],
                                  ssem.at[p]).wait()

    plsc.subcore_barrier()
    pltpu.sync_copy(acc.at[pl.ds(sid * WPT, WPT)],
                    out_hbm.at[c, pl.ds(sid * WPT, WPT)])


def _deg_call(row2):
    return pl.kernel(
        _deg_kernel,
        out_type=jax.ShapeDtypeStruct((NSC, ACC_ROWS, 16), jnp.float32),
        mesh=_mesh(),
        scratch_types=[
            pltpu.VMEM((2, KCH, UNIT), jnp.int32),
            pltpu.VMEM((UNIT, 16), jnp.float32),
            pltpu.VMEM((16, 16), jnp.float32),
            pltpu.VMEM_SHARED((ACC_ROWS, 16), jnp.float32),
            pltpu.SemaphoreType.DMA((2,)),
        ],
        compiler_params=pltpu.CompilerParams(use_tc_tiling_on_sc=False),
    )(row2)


WROWS = UNIT * FH // 128


def _msg_kernel(col2_hbm, row2_hbm, w_hbm, tab_hbm, out_hbm,
                cbig, rbig, gbuf, wbuf, zb, acc, gsem, wsem, ssem, isem):
    c = lax.axis_index("c")
    sid = lax.axis_index("s")

    _zero_acc(acc, zb, sid, FH)
    plsc.subcore_barrier()

    tab_c = tab_hbm.at[c]
    w_c = w_hbm.at[c]
    ubase = sid * UNITS

    def issue(u, b):
        ch = u // KCH
        uk = u % KCH
        pltpu.async_copy(tab_c.at[cbig.at[ch % 2, uk]], gbuf.at[b],
                         gsem.at[b])
        pltpu.async_copy(w_c.at[pl.ds((ubase + u) * WROWS, WROWS)],
                         wbuf.at[b], wsem.at[b])

    def drain_scatter(b):
        pltpu.make_async_copy(gbuf.at[b], acc.at[rbig.at[0, 0]],
                              ssem.at[b]).wait()

    AHEAD = NBUF - 1

    pltpu.sync_copy(col2_hbm.at[pl.ds(ubase, KCH)], cbig.at[0])
    pltpu.sync_copy(row2_hbm.at[pl.ds(ubase, KCH)], rbig.at[0])
    for k in range(AHEAD):
        issue(k, k)

    @pl.loop(0, UNITS // NBUF)
    def _(g):
        for k in range(NBUF):
            u = g * NBUF + k
            b = k
            ch = u // KCH
            uk = u % KCH
            par = ch % 2

            @pl.when(u >= 1)
            def _():
                drain_scatter((k + AHEAD) % NBUF)

            @pl.when((uk == 0) & (ch < NCH - 1))
            def _():
                nxt = (ch + 1) * KCH
                pltpu.async_copy(col2_hbm.at[pl.ds(ubase + nxt, KCH)],
                                 cbig.at[1 - par], isem)
                pltpu.async_copy(row2_hbm.at[pl.ds(ubase + nxt, KCH)],
                                 rbig.at[1 - par], isem)

            @pl.when((uk == KCH - AHEAD) & (ch < NCH - 1))
            def _():
                pltpu.make_async_copy(col2_hbm.at[pl.ds(ubase, KCH)],
                                      cbig.at[0], isem).wait()
                pltpu.make_async_copy(row2_hbm.at[pl.ds(ubase, KCH)],
                                      rbig.at[0], isem).wait()

            pltpu.make_async_copy(tab_c.at[cbig.at[0, 0]], gbuf.at[b],
                                  gsem.at[b]).wait()
            pltpu.make_async_copy(w_c.at[pl.ds(0, WROWS)], wbuf.at[b],
                                  wsem.at[b]).wait()

            @pl.loop(0, WROWS, unroll=2)
            def _(r):
                for j in range(8):
                    e = 4 * r + j // 2
                    gsl = pl.ds((j % 2) * 16, 16)
                    wsl = pl.ds(j * 16, 16)
                    gbuf[b, e, gsl] = gbuf[b, e, gsl] * wbuf[b, r, wsl]

            pltpu.async_copy(gbuf.at[b], acc.at[rbig.at[par, uk]],
                             ssem.at[b], add=True)

            @pl.when(u + AHEAD < UNITS)
            def _():
                issue(u + AHEAD, (k + AHEAD) % NBUF)

    drain_scatter((UNITS - 1) % NBUF)
    plsc.subcore_barrier()
    pltpu.sync_copy(acc.at[pl.ds(sid * WPT, WPT)],
                    out_hbm.at[c, pl.ds(sid * WPT, WPT)])


def _msg_call(col2, row2, w, slin):
    return pl.kernel(
        _msg_kernel,
        out_type=jax.ShapeDtypeStruct((NSC, ACC_ROWS, FH), jnp.float32),
        mesh=_mesh(),
        scratch_types=[
            pltpu.VMEM((2, KCH, UNIT), jnp.int32),
            pltpu.VMEM((2, KCH, UNIT), jnp.int32),
            pltpu.VMEM((NBUF, UNIT, FH), jnp.float32),
            pltpu.VMEM((NBUF, WROWS, 128), jnp.float32),
            pltpu.VMEM((16, FH), jnp.float32),
            pltpu.VMEM_SHARED((ACC_ROWS, FH), jnp.float32),
            pltpu.SemaphoreType.DMA((NBUF,)),
            pltpu.SemaphoreType.DMA((NBUF,)),
            pltpu.SemaphoreType.DMA((NBUF,)),
            pltpu.SemaphoreType.DMA,
        ],
        compiler_params=pltpu.CompilerParams(use_tc_tiling_on_sc=False),
    )(col2, row2, w, slin)


def kernel(x, pos, batch, edge_index, edge_weights, lig_flag, chains, params):
    del pos, lig_flag, chains
    row = edge_index[0]
    col = edge_index[1]
    row2 = jnp.pad(row, (0, E_PAD - E),
                   constant_values=N).reshape(E_PAD // UNIT, UNIT)
    col2 = jnp.pad(col, (0, E_PAD - E),
                   constant_values=0).reshape(E_PAD // UNIT, UNIT)
    d4 = jnp.pad(edge_weights, (0, E_PAD - E),
                 constant_values=1.0).reshape(E_PAD // 4, 4)
    x2 = x.reshape(N, 1)
    batch2 = batch.reshape(N, 1)

    p = params
    emb_pad = jnp.pad(p['emb'], ((0, 128 - p['emb'].shape[0]), (0, 0)))
    layers = p['layers']

    def r1(v):
        return v.reshape(1, -1)

    s2, slin = _embed_call(x2, emb_pad, layers[0]['lin'])

    deg_out = _deg_call(row2)
    deg2 = deg_out[0, :N, 0:1]

    rmat = jnp.repeat(jnp.eye(4, dtype=jnp.float32), NUM_RADIAL, axis=1)
    blkdiag = lambda m: jnp.kron(jnp.eye(4, dtype=jnp.float32), m)

    zlin = jnp.zeros((SDIM, SDIM), jnp.float32)
    for li, lp in enumerate(layers):
        w = _edge_w_call(
            d4, rmat,
            blkdiag(lp['fW1']), r1(jnp.tile(lp['fb1'], 4)),
            blkdiag(lp['fW2'][:, :FH]), r1(jnp.tile(lp['fb2'][:FH], 4)),
            blkdiag(lp['fW2'][:, FH:]), r1(jnp.tile(lp['fb2'][FH:], 4)))
        agg_out = _msg_call(col2, row2, w, slin)
        lin_next = layers[li + 1]['lin'] if li + 1 < DEPTH else zlin
        s2, slin = _update_call(s2, agg_out, deg2, lp['uW1'],
                                r1(lp['ub1']), lp['uW2'], r1(lp['ub2']),
                                lin_next)

    gsum, gcnt = _final_call(s2, batch2, r1(p['ln_g']), r1(p['ln_b']),
                             p['post_lin'])

    a2W_pad = jnp.pad(p['a2W'], ((0, 0), (0, 128 - p['a2W'].shape[1])))
    a2b_pad = jnp.pad(p['a2b'], (0, 128 - p['a2b'].shape[0])).reshape(1, 128)
    out = _head_call(gsum, gcnt, p['d1W'], r1(p['d1b']), p['d2W'], r1(p['d2b']),
                     p['a1W'], r1(p['a1b']), a2W_pad, a2b_pad)
    return out[:, :1]


# speedup vs baseline: 4.1517x; 1.2606x over previous
import functools
import jax
import jax.numpy as jnp
from jax import lax
from jax.experimental import pallas as pl
from jax.experimental.pallas import tpu as pltpu
from jax.experimental.pallas import tpu_sc as plsc

N = 50000
E = 800000
SDIM = 64
NUM_RADIAL = 32
DEPTH = 3
CUTOFF = 5.0
G = 8

NSC = 2
NSUB = 16
FH = SDIM // NSC
UNIT = 64
EPT = 51200
E_PAD = NSUB * EPT
UNITS = EPT // UNIT
KCH = 16
NCH = UNITS // KCH
NBUF = 5
ACC_ROWS = 50176
ZCH = ACC_ROWS // NSUB // 16
WPT = ACC_ROWS // NSUB
BN = 2000
BE = 4096


def _silu(v):
    return v / (1.0 + jnp.exp(-v))


def _sinpi(x):
    y = x * 0.5
    f = y - jnp.round(y)
    t = f * (2.0 * jnp.pi)
    t2 = t * t
    p = 1.0 / 6227020800.0
    p = p * t2 - 1.0 / 39916800.0
    p = p * t2 + 1.0 / 362880.0
    p = p * t2 - 1.0 / 5040.0
    p = p * t2 + 1.0 / 120.0
    p = p * t2 - 1.0 / 6.0
    p = p * t2 + 1.0
    return t * p


def _nblock(feat):
    return pl.BlockSpec((BN, feat), lambda i: (i, 0))


def _wblock(r, cdim=SDIM):
    return pl.BlockSpec((r, cdim), lambda i: (0, 0))


def _split_spec():
    return pl.BlockSpec((NSC, BN, FH), lambda i: (0, i, 0))


def _embed_body(x_ref, emb_ref, lin_ref, s_ref, slin_ref):
    xb = x_ref[...]
    iota = lax.broadcasted_iota(jnp.int32, (BN, 128), 1)
    oh = (iota == xb).astype(jnp.float32)
    s = jnp.dot(oh, emb_ref[...], preferred_element_type=jnp.float32)
    s_ref[...] = s
    sl = jnp.dot(s, lin_ref[...], preferred_element_type=jnp.float32)
    slin_ref[0] = sl[:, :FH]
    slin_ref[1] = sl[:, FH:]


def _embed_call(x2, emb_pad, lin0):
    return pl.pallas_call(
        _embed_body,
        grid=(N // BN,),
        in_specs=[_nblock(1), _wblock(128), _wblock(SDIM)],
        out_specs=[_nblock(SDIM), _split_spec()],
        out_shape=[
            jax.ShapeDtypeStruct((N, SDIM), jnp.float32),
            jax.ShapeDtypeStruct((NSC, N, FH), jnp.float32),
        ],
    )(x2, emb_pad, lin0)


BQ = BE // 4


def _edge_w_body(d_ref, rmat_ref, W1_ref, b1_ref, W2a_ref, b2a_ref,
                 W2b_ref, b2b_ref, w_ref):
    dg = d_ref[...]
    dd = jnp.dot(dg, rmat_ref[...], preferred_element_type=jnp.float32)
    n = (lax.broadcasted_iota(jnp.int32, (BQ, 128), 1) % NUM_RADIAL + 1
         ).astype(jnp.float32)
    rbf = jnp.sqrt(2.0 / CUTOFF) * _sinpi(n * (1.0 / CUTOFF) * dd) / dd
    envg = 0.5 * (_sinpi(dg * (1.0 / CUTOFF) + 0.5) + 1.0)
    envg = envg * (dg < CUTOFF).astype(jnp.float32)
    env = jnp.dot(envg, rmat_ref[...], preferred_element_type=jnp.float32)
    h = _silu(jnp.dot(rbf, W1_ref[...], preferred_element_type=jnp.float32)
              + b1_ref[...])
    wa = jnp.dot(h, W2a_ref[...], preferred_element_type=jnp.float32) + b2a_ref[...]
    wb = jnp.dot(h, W2b_ref[...], preferred_element_type=jnp.float32) + b2b_ref[...]
    w_ref[0] = wa * env
    w_ref[1] = wb * env


def _edge_w_call(d4, rmat, W1blk, b1t, W2a, b2a, W2b, b2b):
    grid = E_PAD // BE
    return pl.pallas_call(
        _edge_w_body,
        grid=(grid,),
        in_specs=[
            pl.BlockSpec((BQ, 4), lambda i: (i, 0)),
            pl.BlockSpec((4, 128), lambda i: (0, 0)),
            pl.BlockSpec((128, 256), lambda i: (0, 0)),
            pl.BlockSpec((1, 256), lambda i: (0, 0)),
            pl.BlockSpec((256, 128), lambda i: (0, 0)),
            pl.BlockSpec((1, 128), lambda i: (0, 0)),
            pl.BlockSpec((256, 128), lambda i: (0, 0)),
            pl.BlockSpec((1, 128), lambda i: (0, 0)),
        ],
        out_specs=pl.BlockSpec((NSC, BQ, 128), lambda i: (0, i, 0)),
        out_shape=jax.ShapeDtypeStruct((NSC, E_PAD // 4, 128), jnp.float32),
    )(d4, rmat, W1blk, b1t, W2a, b2a, W2b, b2b)


def _update_body(s_ref, agg_ref, deg_ref, uW1_ref, ub1_ref, uW2_ref, ub2_ref,
                 lin_ref, s_out_ref, slin_ref):
    deg = jnp.maximum(deg_ref[...], 1.0)
    a = jnp.concatenate([agg_ref[0], agg_ref[1]], axis=-1) / deg
    h = _silu(jnp.dot(a, uW1_ref[...], preferred_element_type=jnp.float32)
              + ub1_ref[...])
    s_new = s_ref[...] + jnp.dot(h, uW2_ref[...],
                                 preferred_element_type=jnp.float32) + ub2_ref[...]
    s_out_ref[...] = s_new
    sl = jnp.dot(s_new, lin_ref[...], preferred_element_type=jnp.float32)
    slin_ref[0] = sl[:, :FH]
    slin_ref[1] = sl[:, FH:]


def _update_call(s2, agg_out, deg2, uW1, ub1, uW2, ub2, lin_next):
    return pl.pallas_call(
        _update_body,
        grid=(N // BN,),
        in_specs=[
            _nblock(SDIM),
            pl.BlockSpec((NSC, BN, FH), lambda i: (0, i, 0)),
            _nblock(1),
            _wblock(SDIM), _wblock(1), _wblock(SDIM), _wblock(1),
            _wblock(SDIM),
        ],
        out_specs=[_nblock(SDIM), _split_spec()],
        out_shape=[
            jax.ShapeDtypeStruct((N, SDIM), jnp.float32),
            jax.ShapeDtypeStruct((NSC, N, FH), jnp.float32),
        ],
    )(s2, agg_out, deg2, uW1, ub1, uW2, ub2, lin_next)


def _final_body(s_ref, batch_ref, lng_ref, lnb_ref, post_ref,
                gsum_ref, gcnt_ref):
    @pl.when(pl.program_id(0) == 0)
    def _():
        gsum_ref[...] = jnp.zeros_like(gsum_ref)
        gcnt_ref[...] = jnp.zeros_like(gcnt_ref)

    s = s_ref[...]
    mu = jnp.mean(s, axis=-1, keepdims=True)
    xc = s - mu
    var = jnp.mean(xc * xc, axis=-1, keepdims=True)
    sn = xc / jnp.sqrt(var + 1e-5) * lng_ref[...] + lnb_ref[...]
    p = jnp.dot(sn, post_ref[...], preferred_element_type=jnp.float32)
    bb = batch_ref[...]
    gio = lax.broadcasted_iota(jnp.int32, (BN, G), 1)
    oh = (gio == bb).astype(jnp.float32)
    part = lax.dot_general(oh, p, (((0,), (0,)), ((), ())),
                           preferred_element_type=jnp.float32)
    cnt = lax.dot_general(oh, jnp.ones((BN, SDIM), jnp.float32),
                          (((0,), (0,)), ((), ())),
                          preferred_element_type=jnp.float32)
    gsum_ref[...] += part
    gcnt_ref[...] += cnt


def _final_call(s2, batch2, lng, lnb, post_lin):
    return pl.pallas_call(
        _final_body,
        grid=(N // BN,),
        in_specs=[
            _nblock(SDIM), _nblock(1),
            _wblock(1), _wblock(1), _wblock(SDIM),
        ],
        out_specs=[
            pl.BlockSpec((G, SDIM), lambda i: (0, 0)),
            pl.BlockSpec((G, SDIM), lambda i: (0, 0)),
        ],
        out_shape=[
            jax.ShapeDtypeStruct((G, SDIM), jnp.float32),
            jax.ShapeDtypeStruct((G, SDIM), jnp.float32),
        ],
    )(s2, batch2, lng, lnb, post_lin)


def _head_body(gsum_ref, gcnt_ref, d1W_ref, d1b_ref, d2W_ref, d2b_ref,
               a1W_ref, a1b_ref, a2W_ref, a2b_ref, out_ref):
    y = gsum_ref[...] / jnp.maximum(gcnt_ref[...], 1.0)
    y = _silu(jnp.dot(y, d1W_ref[...], preferred_element_type=jnp.float32)
              + d1b_ref[...])
    y = jnp.dot(y, d2W_ref[...], preferred_element_type=jnp.float32) + d2b_ref[...]
    a = _silu(jnp.dot(y, a1W_ref[...], preferred_element_type=jnp.float32)
              + a1b_ref[...])
    out_ref[...] = jnp.dot(a, a2W_ref[...],
                           preferred_element_type=jnp.float32) + a2b_ref[...]


def _head_call(gsum, gcnt, d1W, d1b, d2W, d2b, a1W, a1b, a2W_pad, a2b_pad):
    return pl.pallas_call(
        _head_body,
        out_shape=jax.ShapeDtypeStruct((G, 128), jnp.float32),
    )(gsum, gcnt, d1W, d1b, d2W, d2b, a1W, a1b, a2W_pad, a2b_pad)


@functools.cache
def _mesh():
    return plsc.VectorSubcoreMesh(core_axis_name="c", subcore_axis_name="s",
                                  num_cores=NSC, num_subcores=NSUB)


def _zero_acc(acc, zb, sid, width):
    zeros16 = jnp.zeros((16,), jnp.float32)

    @pl.loop(0, 16)
    def _(e):
        for j in range(width // 16):
            zb[e, pl.ds(j * 16, 16)] = zeros16

    @pl.loop(0, ZCH)
    def _(k):
        pltpu.sync_copy(zb, acc.at[pl.ds((sid * ZCH + k) * 16, 16)])


def _deg_kernel(row2_hbm, out_hbm, rbig, onesb, zb, acc, ssem):
    c = lax.axis_index("c")
    sid = lax.axis_index("s")

    ones16 = jnp.ones((16,), jnp.float32)

    @pl.loop(0, UNIT)
    def _(e):
        onesb[e, :] = ones16

    _zero_acc(acc, zb, sid, 16)
    plsc.subcore_barrier()

    @pl.loop(0, NCH)
    def _(ch):
        p = ch % 2

        @pl.when(ch >= 2)
        def _():
            @pl.loop(0, KCH)
            def _(u):
                pltpu.make_async_copy(onesb, acc.at[rbig.at[p, 0]],
                                      ssem.at[p]).wait()

        pltpu.sync_copy(
            row2_hbm.at[pl.ds(sid * UNITS + ch * KCH, KCH)], rbig.at[p])

        @pl.loop(0, KCH)
        def _(u):
            pltpu.async_copy(onesb, acc.at[rbig.at[p, u]], ssem.at[p],
                             add=True)

    for p in range(2):
        @pl.loop(0, KCH)
        def _(u):
            pltpu.make_async_copy(onesb, acc.at[rbig.at[p, 0]],
                                  ssem.at[p]).wait()

    plsc.subcore_barrier()
    pltpu.sync_copy(acc.at[pl.ds(sid * WPT, WPT)],
                    out_hbm.at[c, pl.ds(sid * WPT, WPT)])


def _deg_call(row2):
    return pl.kernel(
        _deg_kernel,
        out_type=jax.ShapeDtypeStruct((NSC, ACC_ROWS, 16), jnp.float32),
        mesh=_mesh(),
        scratch_types=[
            pltpu.VMEM((2, KCH, UNIT), jnp.int32),
            pltpu.VMEM((UNIT, 16), jnp.float32),
            pltpu.VMEM((16, 16), jnp.float32),
            pltpu.VMEM_SHARED((ACC_ROWS, 16), jnp.float32),
            pltpu.SemaphoreType.DMA((2,)),
        ],
        compiler_params=pltpu.CompilerParams(use_tc_tiling_on_sc=False),
    )(row2)


WROWS = UNIT * FH // 128


def _msg_kernel(col2_hbm, row2_hbm, w_hbm, tab_hbm, out_hbm,
                cbig, rbig, gbuf, wbuf, zb, acc, gsem, wsem, ssem, isem):
    c = lax.axis_index("c")
    sid = lax.axis_index("s")

    _zero_acc(acc, zb, sid, FH)
    plsc.subcore_barrier()

    tab_c = tab_hbm.at[c]
    w_c = w_hbm.at[c]
    ubase = sid * UNITS

    def issue(u, b):
        ch = u // KCH
        uk = u % KCH
        pltpu.async_copy(tab_c.at[cbig.at[ch % 2, uk]], gbuf.at[b],
                         gsem.at[b])
        pltpu.async_copy(w_c.at[pl.ds((ubase + u) * WROWS, WROWS)],
                         wbuf.at[b], wsem.at[b])

    def drain_scatter(b):
        pltpu.make_async_copy(gbuf.at[b], acc.at[rbig.at[0, 0]],
                              ssem.at[b]).wait()

    AHEAD = NBUF - 1

    pltpu.sync_copy(col2_hbm.at[pl.ds(ubase, KCH)], cbig.at[0])
    pltpu.sync_copy(row2_hbm.at[pl.ds(ubase, KCH)], rbig.at[0])
    for k in range(AHEAD):
        issue(k, k)

    @pl.loop(0, UNITS // NBUF)
    def _(g):
        for k in range(NBUF):
            u = g * NBUF + k
            b = k
            ch = u // KCH
            uk = u % KCH
            par = ch % 2

            @pl.when(u >= 1)
            def _():
                drain_scatter((k + AHEAD) % NBUF)

            @pl.when((uk == 0) & (ch < NCH - 1))
            def _():
                nxt = (ch + 1) * KCH
                pltpu.async_copy(col2_hbm.at[pl.ds(ubase + nxt, KCH)],
                                 cbig.at[1 - par], isem)
                pltpu.async_copy(row2_hbm.at[pl.ds(ubase + nxt, KCH)],
                                 rbig.at[1 - par], isem)

            @pl.when((uk == KCH - AHEAD) & (ch < NCH - 1))
            def _():
                pltpu.make_async_copy(col2_hbm.at[pl.ds(ubase, KCH)],
                                      cbig.at[0], isem).wait()
                pltpu.make_async_copy(row2_hbm.at[pl.ds(ubase, KCH)],
                                      rbig.at[0], isem).wait()

            pltpu.make_async_copy(tab_c.at[cbig.at[0, 0]], gbuf.at[b],
                                  gsem.at[b]).wait()
            pltpu.make_async_copy(w_c.at[pl.ds(0, WROWS)], wbuf.at[b],
                                  wsem.at[b]).wait()

            @pl.loop(0, WROWS, unroll=2)
            def _(r):
                for j in range(8):
                    e = 4 * r + j // 2
                    gsl = pl.ds((j % 2) * 16, 16)
                    wsl = pl.ds(j * 16, 16)
                    gbuf[b, e, gsl] = gbuf[b, e, gsl] * wbuf[b, r, wsl]

            pltpu.async_copy(gbuf.at[b], acc.at[rbig.at[par, uk]],
                             ssem.at[b], add=True)

            @pl.when(u + AHEAD < UNITS)
            def _():
                issue(u + AHEAD, (k + AHEAD) % NBUF)

    drain_scatter((UNITS - 1) % NBUF)
    plsc.subcore_barrier()
    pltpu.sync_copy(acc.at[pl.ds(sid * WPT, WPT)],
                    out_hbm.at[c, pl.ds(sid * WPT, WPT)])


def _msg_call(col2, row2, w, slin):
    return pl.kernel(
        _msg_kernel,
        out_type=jax.ShapeDtypeStruct((NSC, ACC_ROWS, FH), jnp.float32),
        mesh=_mesh(),
        scratch_types=[
            pltpu.VMEM((2, KCH, UNIT), jnp.int32),
            pltpu.VMEM((2, KCH, UNIT), jnp.int32),
            pltpu.VMEM((NBUF, UNIT, FH), jnp.float32),
            pltpu.VMEM((NBUF, WROWS, 128), jnp.float32),
            pltpu.VMEM((16, FH), jnp.float32),
            pltpu.VMEM_SHARED((ACC_ROWS, FH), jnp.float32),
            pltpu.SemaphoreType.DMA((NBUF,)),
            pltpu.SemaphoreType.DMA((NBUF,)),
            pltpu.SemaphoreType.DMA((NBUF,)),
            pltpu.SemaphoreType.DMA,
        ],
        compiler_params=pltpu.CompilerParams(use_tc_tiling_on_sc=False),
    )(col2, row2, w, slin)


def kernel(x, pos, batch, edge_index, edge_weights, lig_flag, chains, params):
    del pos, lig_flag, chains
    row = edge_index[0]
    col = edge_index[1]
    row2 = jnp.pad(row, (0, E_PAD - E),
                   constant_values=N).reshape(E_PAD // UNIT, UNIT)
    col2 = jnp.pad(col, (0, E_PAD - E),
                   constant_values=0).reshape(E_PAD // UNIT, UNIT)
    d4 = jnp.pad(edge_weights, (0, E_PAD - E),
                 constant_values=1.0).reshape(E_PAD // 4, 4)
    x2 = x.reshape(N, 1)
    batch2 = batch.reshape(N, 1)

    p = params
    emb_pad = jnp.pad(p['emb'], ((0, 128 - p['emb'].shape[0]), (0, 0)))
    layers = p['layers']

    def r1(v):
        return v.reshape(1, -1)

    s2, slin = _embed_call(x2, emb_pad, layers[0]['lin'])

    deg_out = _deg_call(row2)
    deg2 = deg_out[0, :N, 0:1]

    rmat = jnp.repeat(jnp.eye(4, dtype=jnp.float32), NUM_RADIAL, axis=1)
    blkdiag = lambda m: jnp.kron(jnp.eye(4, dtype=jnp.float32), m)

    zlin = jnp.zeros((SDIM, SDIM), jnp.float32)
    for li, lp in enumerate(layers):
        w = _edge_w_call(
            d4, rmat,
            blkdiag(lp['fW1']), r1(jnp.tile(lp['fb1'], 4)),
            blkdiag(lp['fW2'][:, :FH]), r1(jnp.tile(lp['fb2'][:FH], 4)),
            blkdiag(lp['fW2'][:, FH:]), r1(jnp.tile(lp['fb2'][FH:], 4)))
        agg_out = _msg_call(col2, row2, w, slin)
        lin_next = layers[li + 1]['lin'] if li + 1 < DEPTH else zlin
        s2, slin = _update_call(s2, agg_out, deg2, lp['uW1'],
                                r1(lp['ub1']), lp['uW2'], r1(lp['ub2']),
                                lin_next)

    gsum, gcnt = _final_call(s2, batch2, r1(p['ln_g']), r1(p['ln_b']),
                             p['post_lin'])

    a2W_pad = jnp.pad(p['a2W'], ((0, 0), (0, 128 - p['a2W'].shape[1])))
    a2b_pad = jnp.pad(p['a2b'], (0, 128 - p['a2b'].shape[0])).reshape(1, 128)
    out = _head_call(gsum, gcnt, p['d1W'], r1(p['d1b']), p['d2W'], r1(p['d2b']),
                     p['a1W'], r1(p['a1b']), a2W_pad, a2b_pad)
    return out[:, :1]
